# baseline (speedup 1.0000x reference)
import functools

import jax
import jax.numpy as jnp
from jax import lax
from jax.experimental import pallas as pl
from jax.experimental.pallas import tpu as pltpu
from jax.experimental.pallas import tpu_sc as plsc

_N, _E, _D, _H, _C = 10000, 320000, 128, 128, 64
_NC, _NS = 2, 16
_NW = _NC * _NS
_UNIT = 128
_NUNITS = _E // _UNIT
_BASE = _NUNITS // _NW
_EXTRA = _NUNITS - _BASE * _NW
_NUNITS_PAD = 2560
_NP = 10240
_RPS = _NP // _NS
_DEGW = 128

_mesh = lambda: plsc.VectorSubcoreMesh(core_axis_name="c", subcore_axis_name="s")


def _zero16():
    return jnp.zeros((16,), jnp.float32)


def _zero_acc_slice(zrows, acc, base):
    for t in range(_RPS // _UNIT):
        pltpu.sync_copy(zrows, acc.at[pl.ds(base + t * _UNIT, _UNIT)])


def _make_deg():
    @functools.partial(
        pl.kernel,
        out_type=jax.ShapeDtypeStruct((_NC, _NP, _DEGW), jnp.float32),
        mesh=_mesh(),
        scratch_types=[
            pltpu.VMEM((1, _UNIT), jnp.int32),
            pltpu.VMEM((_UNIT, _DEGW), jnp.float32),
            pltpu.VMEM((_UNIT, _DEGW), jnp.float32),
            pltpu.VMEM_SHARED((_NP, _DEGW), jnp.float32),
        ],
    )
    def deg_kernel(dsts, out, dstv, onesv, zv, acc):
        cid = lax.axis_index("c")
        sid = lax.axis_index("s")
        wid = sid * _NC + cid
        base = sid * _RPS

        ones16 = jnp.ones((16,), jnp.float32)
        z16 = _zero16()

        def fill(r, carry):
            for cidx in range(_DEGW // 16):
                onesv[r, pl.ds(cidx * 16, 16)] = ones16
                zv[r, pl.ds(cidx * 16, 16)] = z16
            return carry

        lax.fori_loop(0, _UNIT, fill, 0)
        _zero_acc_slice(zv, acc, base)
        plsc.subcore_barrier()

        def body(u):
            off = u * _UNIT
            pltpu.sync_copy(dsts.at[pl.ds(off, _UNIT)], dstv.at[0])
            pltpu.sync_copy(onesv, acc.at[dstv.at[0]], add=True)

        def loop(i, carry):
            body(wid + i * _NW)
            return carry

        lax.fori_loop(0, _BASE, loop, 0)

        @pl.when(wid < _EXTRA)
        def _():
            body(wid + _BASE * _NW)

        plsc.subcore_barrier()
        pltpu.sync_copy(acc.at[pl.ds(base, _RPS)],
                        out.at[cid, pl.ds(base, _RPS)])

    return deg_kernel


_UPW = _NUNITS_PAD // _NW
_UPP = _UPW // 2


def _make_agg(F):
    @functools.partial(
        pl.kernel,
        out_type=jax.ShapeDtypeStruct((_NC, _NP, F), jnp.float32),
        mesh=_mesh(),
        scratch_types=[
            pltpu.VMEM((_UPP, _UNIT), jnp.int32),
            pltpu.VMEM((_UPP, _UNIT), jnp.int32),
            pltpu.VMEM((_UNIT, F), jnp.float32),
            pltpu.VMEM((_UNIT, F), jnp.float32),
            pltpu.SemaphoreType.DMA,
            pltpu.SemaphoreType.DMA,
            pltpu.VMEM_SHARED((_NP, F), jnp.float32),
        ],
    )
    def agg_kernel(src2d, dst2d, h, out, srcall, dstall, rows0, rows1,
                   g0, g1, acc):
        cid = lax.axis_index("c")
        sid = lax.axis_index("s")
        wid = sid * _NC + cid
        base = sid * _RPS
        u0 = wid * _UPW

        z16 = _zero16()

        def zfill(r, carry):
            for cidx in range(F // 16):
                rows0[r, pl.ds(cidx * 16, 16)] = z16
            return carry

        lax.fori_loop(0, _UNIT, zfill, 0)
        _zero_acc_slice(rows0, acc, base)
        plsc.subcore_barrier()

        def gather(u, buf, sem):
            return pltpu.async_copy(h.at[srcall.at[u]], buf, sem)

        def gwait(u, buf, sem):
            pltpu.make_async_copy(h.at[srcall.at[u]], buf, sem).wait()

        def scatter(u, buf):
            pltpu.sync_copy(buf, acc.at[dstall.at[u]], add=True)

        for p in range(2):
            pltpu.sync_copy(src2d.at[pl.ds(u0 + p * _UPP, _UPP)], srcall)
            pltpu.sync_copy(dst2d.at[pl.ds(u0 + p * _UPP, _UPP)], dstall)
            gather(0, rows0, g0)

            def step(i, carry):
                u = 2 * i
                gather(u + 1, rows1, g1)
                gwait(u, rows0, g0)
                scatter(u, rows0)

                @pl.when(u + 2 < _UPP)
                def _():
                    gather(u + 2, rows0, g0)

                gwait(u + 1, rows1, g1)
                scatter(u + 1, rows1)
                return carry

            lax.fori_loop(0, _UPP // 2, step, 0)

        plsc.subcore_barrier()
        pltpu.sync_copy(acc.at[pl.ds(base, _RPS)],
                        out.at[cid, pl.ds(base, _RPS)])

    return agg_kernel


_deg_call = _make_deg()
_agg_h = _make_agg(_H)

_BLK = 1000
_GRID = _N // _BLK


def _dinv_of(d_ref):
    s = d_ref[0, :, 0:1] + d_ref[1, :, 0:1] + 1.0
    return lax.rsqrt(s)


def _mm1(x, W1, deg):
    def body(x_ref, w_ref, d_ref, o_ref):
        dinv = _dinv_of(d_ref)
        h = jnp.dot(x_ref[...], w_ref[...], preferred_element_type=jnp.float32)
        o_ref[...] = h * dinv

    return pl.pallas_call(
        body,
        grid=(_GRID,),
        in_specs=[
            pl.BlockSpec((_BLK, _D), lambda i: (i, 0)),
            pl.BlockSpec((_D, _H), lambda i: (0, 0)),
            pl.BlockSpec((_NC, _BLK, _DEGW), lambda i: (0, i, 0)),
        ],
        out_specs=pl.BlockSpec((_BLK, _H), lambda i: (i, 0)),
        out_shape=jax.ShapeDtypeStruct((_N, _H), jnp.float32),
    )(x, W1, deg)


def _mm2(acc1, h1s, deg, x, b1, W2):
    def body(a_ref, h_ref, d_ref, x_ref, b_ref, w_ref, o_ref):
        dinv = _dinv_of(d_ref)
        agg = a_ref[0] + a_ref[1] + h_ref[...]
        out1 = agg * dinv + b_ref[...]
        z = jnp.maximum(out1, 0.0) + x_ref[...]
        h2 = jnp.dot(z, w_ref[...], preferred_element_type=jnp.float32)
        o_ref[...] = jnp.concatenate([h2 * dinv, jnp.zeros_like(h2)], axis=1)

    return pl.pallas_call(
        body,
        grid=(_GRID,),
        in_specs=[
            pl.BlockSpec((_NC, _BLK, _H), lambda i: (0, i, 0)),
            pl.BlockSpec((_BLK, _H), lambda i: (i, 0)),
            pl.BlockSpec((_NC, _BLK, _DEGW), lambda i: (0, i, 0)),
            pl.BlockSpec((_BLK, _D), lambda i: (i, 0)),
            pl.BlockSpec((_H,), lambda i: (0,)),
            pl.BlockSpec((_H, _C), lambda i: (0, 0)),
        ],
        out_specs=pl.BlockSpec((_BLK, 2 * _C), lambda i: (i, 0)),
        out_shape=jax.ShapeDtypeStruct((_N, 2 * _C), jnp.float32),
    )(acc1, h1s, deg, x, b1, W2)


def _mm3(acc2, h2s, deg, b2):
    def body(a_ref, h_ref, d_ref, b_ref, o_ref):
        dinv = _dinv_of(d_ref)
        agg = a_ref[0, :, : _C] + a_ref[1, :, : _C] + h_ref[:, : _C]
        o_ref[...] = agg * dinv + b_ref[...]

    return pl.pallas_call(
        body,
        grid=(_GRID,),
        in_specs=[
            pl.BlockSpec((_NC, _BLK, 2 * _C), lambda i: (0, i, 0)),
            pl.BlockSpec((_BLK, 2 * _C), lambda i: (i, 0)),
            pl.BlockSpec((_NC, _BLK, _DEGW), lambda i: (0, i, 0)),
            pl.BlockSpec((_C,), lambda i: (0,)),
        ],
        out_specs=pl.BlockSpec((_BLK, _C), lambda i: (i, 0)),
        out_shape=jax.ShapeDtypeStruct((_N, _C), jnp.float32),
    )(acc2, h2s, deg, b2)


def kernel(x, edge_index, W1, b1, W2, b2):
    ei = edge_index.astype(jnp.int32)
    srcs = ei[0]
    dsts = ei[1]
    npad = _NUNITS_PAD * _UNIT - _E
    src2d = jnp.concatenate(
        [srcs, jnp.zeros((npad,), jnp.int32)]).reshape(_NUNITS_PAD, _UNIT)
    dst2d = jnp.concatenate(
        [dsts, _N + (jnp.arange(npad, dtype=jnp.int32) % (_NP - _N))]
    ).reshape(_NUNITS_PAD, _UNIT)
    deg = _deg_call(dsts)
    h1s = _mm1(x, W1, deg)
    acc1 = _agg_h(src2d, dst2d, h1s)
    h2s = _mm2(acc1, h1s, deg, x, b1, W2)
    acc2 = _agg_h(src2d, dst2d, h2s)
    y = _mm3(acc2, h2s, deg, b2)
    return y


# speedup vs baseline: 2.8944x; 2.8944x over previous
import functools

import jax
import jax.numpy as jnp
from jax import lax
from jax.experimental import pallas as pl
from jax.experimental.pallas import tpu as pltpu
from jax.experimental.pallas import tpu_sc as plsc

_N, _E, _D, _H, _C = 10000, 320000, 128, 128, 64
_NC, _NS = 2, 16
_NW = _NC * _NS
_UNIT = 128
_NUNITS = _E // _UNIT
_BASE = _NUNITS // _NW
_EXTRA = _NUNITS - _BASE * _NW
_NUNITS_PAD = 2560
_NP = 10240
_RPS = _NP // _NS
_DEGW = 128

_mesh = lambda: plsc.VectorSubcoreMesh(core_axis_name="c", subcore_axis_name="s")


def _zero16():
    return jnp.zeros((16,), jnp.float32)


def _zero_acc_slice(zrows, acc, base):
    for t in range(_RPS // _UNIT):
        pltpu.sync_copy(zrows, acc.at[pl.ds(base + t * _UNIT, _UNIT)])


def _make_deg():
    @functools.partial(
        pl.kernel,
        out_type=jax.ShapeDtypeStruct((_NC, _NP, _DEGW), jnp.float32),
        mesh=_mesh(),
        scratch_types=[
            pltpu.VMEM((1, _UNIT), jnp.int32),
            pltpu.VMEM((_UNIT, _DEGW), jnp.float32),
            pltpu.VMEM((_UNIT, _DEGW), jnp.float32),
            pltpu.VMEM_SHARED((_NP, _DEGW), jnp.float32),
        ],
    )
    def deg_kernel(dsts, out, dstv, onesv, zv, acc):
        cid = lax.axis_index("c")
        sid = lax.axis_index("s")
        wid = sid * _NC + cid
        base = sid * _RPS

        ones16 = jnp.ones((16,), jnp.float32)
        z16 = _zero16()

        def fill(r, carry):
            for cidx in range(_DEGW // 16):
                onesv[r, pl.ds(cidx * 16, 16)] = ones16
                zv[r, pl.ds(cidx * 16, 16)] = z16
            return carry

        lax.fori_loop(0, _UNIT, fill, 0)
        _zero_acc_slice(zv, acc, base)
        plsc.subcore_barrier()

        def body(u):
            off = u * _UNIT
            pltpu.sync_copy(dsts.at[pl.ds(off, _UNIT)], dstv.at[0])
            pltpu.sync_copy(onesv, acc.at[dstv.at[0]], add=True)

        def loop(i, carry):
            body(wid + i * _NW)
            return carry

        lax.fori_loop(0, _BASE, loop, 0)

        @pl.when(wid < _EXTRA)
        def _():
            body(wid + _BASE * _NW)

        plsc.subcore_barrier()
        pltpu.sync_copy(acc.at[pl.ds(base, _RPS)],
                        out.at[cid, pl.ds(base, _RPS)])

    return deg_kernel


_UPW = _NUNITS_PAD // _NW
_UPP = _UPW // 2


def _make_agg(F):
    @functools.partial(
        pl.kernel,
        out_type=jax.ShapeDtypeStruct((_NC, _NP, F), jnp.float32),
        mesh=_mesh(),
        scratch_types=[
            pltpu.VMEM((_UPP, _UNIT), jnp.int32),
            pltpu.VMEM((_UPP, _UNIT), jnp.int32),
            pltpu.VMEM((_UNIT, F), jnp.float32),
            pltpu.VMEM((_UNIT, F), jnp.float32),
            pltpu.SemaphoreType.DMA,
            pltpu.SemaphoreType.DMA,
            pltpu.VMEM_SHARED((_NP, F), jnp.float32),
        ],
    )
    def agg_kernel(src2d, dst2d, h, out, srcall, dstall, rows0, rows1,
                   g0, g1, acc):
        cid = lax.axis_index("c")
        sid = lax.axis_index("s")
        wid = sid * _NC + cid
        base = sid * _RPS
        u0 = wid * _UPW

        z16 = _zero16()

        def zfill(r, carry):
            for cidx in range(F // 16):
                rows0[r, pl.ds(cidx * 16, 16)] = z16
            return carry

        lax.fori_loop(0, _UNIT, zfill, 0)
        _zero_acc_slice(rows0, acc, base)
        plsc.subcore_barrier()

        def gather(u, buf, sem):
            return pltpu.async_copy(h.at[srcall.at[u]], buf, sem)

        def gwait(u, buf, sem):
            pltpu.make_async_copy(h.at[srcall.at[u]], buf, sem).wait()

        def scatter(u, buf):
            pltpu.sync_copy(buf, acc.at[dstall.at[u]], add=True)

        for p in range(2):
            pltpu.sync_copy(src2d.at[pl.ds(u0 + p * _UPP, _UPP)], srcall)
            pltpu.sync_copy(dst2d.at[pl.ds(u0 + p * _UPP, _UPP)], dstall)
            gather(0, rows0, g0)

            def step(i, carry):
                u = 2 * i
                gather(u + 1, rows1, g1)
                gwait(u, rows0, g0)
                scatter(u, rows0)

                @pl.when(u + 2 < _UPP)
                def _():
                    gather(u + 2, rows0, g0)

                gwait(u + 1, rows1, g1)
                scatter(u + 1, rows1)
                return carry

            lax.fori_loop(0, _UPP // 2, step, 0)

        plsc.subcore_barrier()
        pltpu.sync_copy(acc.at[pl.ds(base, _RPS)],
                        out.at[cid, pl.ds(base, _RPS)])

    return agg_kernel


_deg_call = _make_deg()
_agg_h = _make_agg(_H)

_BLK = 1000
_GRID = _N // _BLK


def _dinv_of(d_ref):
    s = d_ref[0, :, 0:1] + d_ref[1, :, 0:1] + 1.0
    return lax.rsqrt(s)


def _mm1(x, W1, deg):
    def body(x_ref, w_ref, d_ref, o_ref):
        dinv = _dinv_of(d_ref)
        h = jnp.dot(x_ref[...], w_ref[...], preferred_element_type=jnp.float32)
        o_ref[...] = h * dinv

    return pl.pallas_call(
        body,
        grid=(_GRID,),
        in_specs=[
            pl.BlockSpec((_BLK, _D), lambda i: (i, 0)),
            pl.BlockSpec((_D, _H), lambda i: (0, 0)),
            pl.BlockSpec((_NC, _BLK, _DEGW), lambda i: (0, i, 0)),
        ],
        out_specs=pl.BlockSpec((_BLK, _H), lambda i: (i, 0)),
        out_shape=jax.ShapeDtypeStruct((_N, _H), jnp.float32),
    )(x, W1, deg)


def _mm2(acc1, h1s, deg, x, b1, W2):
    def body(a_ref, h_ref, d_ref, x_ref, b_ref, w_ref, o_ref):
        dinv = _dinv_of(d_ref)
        agg = a_ref[0] + a_ref[1] + h_ref[...]
        out1 = agg * dinv + b_ref[...]
        z = jnp.maximum(out1, 0.0) + x_ref[...]
        h2 = jnp.dot(z, w_ref[...], preferred_element_type=jnp.float32)
        o_ref[...] = jnp.concatenate([h2 * dinv, jnp.zeros_like(h2)], axis=1)

    return pl.pallas_call(
        body,
        grid=(_GRID,),
        in_specs=[
            pl.BlockSpec((_NC, _BLK, _H), lambda i: (0, i, 0)),
            pl.BlockSpec((_BLK, _H), lambda i: (i, 0)),
            pl.BlockSpec((_NC, _BLK, _DEGW), lambda i: (0, i, 0)),
            pl.BlockSpec((_BLK, _D), lambda i: (i, 0)),
            pl.BlockSpec((_H,), lambda i: (0,)),
            pl.BlockSpec((_H, _C), lambda i: (0, 0)),
        ],
        out_specs=pl.BlockSpec((_BLK, 2 * _C), lambda i: (i, 0)),
        out_shape=jax.ShapeDtypeStruct((_N, 2 * _C), jnp.float32),
    )(acc1, h1s, deg, x, b1, W2)


def _mm3(acc2, h2s, deg, b2):
    def body(a_ref, h_ref, d_ref, b_ref, o_ref):
        dinv = _dinv_of(d_ref)
        agg = a_ref[0, :, : _C] + a_ref[1, :, : _C] + h_ref[:, : _C]
        o_ref[...] = agg * dinv + b_ref[...]

    return pl.pallas_call(
        body,
        grid=(_GRID,),
        in_specs=[
            pl.BlockSpec((_NC, _BLK, 2 * _C), lambda i: (0, i, 0)),
            pl.BlockSpec((_BLK, 2 * _C), lambda i: (i, 0)),
            pl.BlockSpec((_NC, _BLK, _DEGW), lambda i: (0, i, 0)),
            pl.BlockSpec((_C,), lambda i: (0,)),
        ],
        out_specs=pl.BlockSpec((_BLK, _C), lambda i: (i, 0)),
        out_shape=jax.ShapeDtypeStruct((_N, _C), jnp.float32),
    )(acc2, h2s, deg, b2)


def kernel(x, edge_index, W1, b1, W2, b2):
    ei = edge_index.astype(jnp.int32)
    srcs = ei[0]
    dsts = ei[1]
    npad = _NUNITS_PAD * _UNIT - _E
    pad_src = (jnp.arange(npad, dtype=jnp.int32) * 131) % _N
    src2d = jnp.concatenate([srcs, pad_src]).reshape(_NUNITS_PAD, _UNIT)
    dst2d = jnp.concatenate(
        [dsts, _N + (jnp.arange(npad, dtype=jnp.int32) % (_NP - _N))]
    ).reshape(_NUNITS_PAD, _UNIT)
    deg = _deg_call(dsts)
    h1s = _mm1(x, W1, deg)
    acc1 = _agg_h(src2d, dst2d, h1s)
    h2s = _mm2(acc1, h1s, deg, x, b1, W2)
    acc2 = _agg_h(src2d, dst2d, h2s)
    y = _mm3(acc2, h2s, deg, b2)
    return y


# speedup vs baseline: 3.1544x; 1.0898x over previous
import functools

import jax
import jax.numpy as jnp
from jax import lax
from jax.experimental import pallas as pl
from jax.experimental.pallas import tpu as pltpu
from jax.experimental.pallas import tpu_sc as plsc

_N, _E, _D, _H, _C = 10000, 320000, 128, 128, 64
_NC, _NS = 2, 16
_NW = _NC * _NS
_UNIT = 128
_NUNITS = _E // _UNIT
_BASE = _NUNITS // _NW
_EXTRA = _NUNITS - _BASE * _NW
_NUNITS_PAD = 2560
_NP = 10240
_RPS = _NP // _NS
_DEGW = 128

_mesh = lambda: plsc.VectorSubcoreMesh(core_axis_name="c", subcore_axis_name="s")


def _zero16():
    return jnp.zeros((16,), jnp.float32)


def _zero_acc_slice(zrows, acc, base):
    for t in range(_RPS // _UNIT):
        pltpu.sync_copy(zrows, acc.at[pl.ds(base + t * _UNIT, _UNIT)])


def _make_deg():
    @functools.partial(
        pl.kernel,
        out_type=jax.ShapeDtypeStruct((_NC, _NP, _DEGW), jnp.float32),
        mesh=_mesh(),
        scratch_types=[
            pltpu.VMEM((40, _UNIT), jnp.int32),
            pltpu.VMEM((_UNIT, _DEGW), jnp.float32),
            pltpu.VMEM((_UNIT, _DEGW), jnp.float32),
            pltpu.SemaphoreType.DMA,
            pltpu.SemaphoreType.DMA,
            pltpu.VMEM_SHARED((_NP, _DEGW), jnp.float32),
        ],
    )
    def deg_kernel(dst2d, out, dstall, onesv, zv, s0, s1, acc):
        cid = lax.axis_index("c")
        sid = lax.axis_index("s")
        wid = sid * _NC + cid
        base = sid * _RPS
        u0 = wid * _UPW

        ones16 = jnp.ones((16,), jnp.float32)
        z16 = _zero16()

        def fill(r, carry):
            for cidx in range(_DEGW // 16):
                onesv[r, pl.ds(cidx * 16, 16)] = ones16
                zv[r, pl.ds(cidx * 16, 16)] = z16
            return carry

        lax.fori_loop(0, _UNIT, fill, 0)
        _zero_acc_slice(zv, acc, base)
        plsc.subcore_barrier()

        def scat(u, sem):
            pltpu.async_copy(onesv, acc.at[dstall.at[u]], sem, add=True)

        def swait(sem):
            pltpu.make_async_copy(onesv, acc.at[dstall.at[0]], sem).wait()

        for p in range(2):
            pltpu.sync_copy(dst2d.at[pl.ds(u0 + p * _UPP, _UPP)], dstall)
            scat(0, s0)
            scat(1, s1)

            def step(i, carry):
                u = 2 * i
                swait(s0)

                @pl.when(u + 2 < _UPP)
                def _():
                    scat(u + 2, s0)

                swait(s1)

                @pl.when(u + 3 < _UPP)
                def _():
                    scat(u + 3, s1)

                return carry

            lax.fori_loop(0, _UPP // 2, step, 0)

        plsc.subcore_barrier()
        pltpu.sync_copy(acc.at[pl.ds(base, _RPS)],
                        out.at[cid, pl.ds(base, _RPS)])

    return deg_kernel


_UPW = _NUNITS_PAD // _NW
_UPP = _UPW // 2


def _make_agg(F):
    @functools.partial(
        pl.kernel,
        out_type=jax.ShapeDtypeStruct((_NC, _NP, F), jnp.float32),
        mesh=_mesh(),
        scratch_types=[
            pltpu.VMEM((_UPP, _UNIT), jnp.int32),
            pltpu.VMEM((_UPP, _UNIT), jnp.int32),
            pltpu.VMEM((_UNIT, F), jnp.float32),
            pltpu.VMEM((_UNIT, F), jnp.float32),
            pltpu.SemaphoreType.DMA,
            pltpu.SemaphoreType.DMA,
            pltpu.VMEM_SHARED((_NP, F), jnp.float32),
        ],
    )
    def agg_kernel(src2d, dst2d, h, out, srcall, dstall, rows0, rows1,
                   g0, g1, acc):
        cid = lax.axis_index("c")
        sid = lax.axis_index("s")
        wid = sid * _NC + cid
        base = sid * _RPS
        u0 = wid * _UPW

        z16 = _zero16()

        def zfill(r, carry):
            for cidx in range(F // 16):
                rows0[r, pl.ds(cidx * 16, 16)] = z16
            return carry

        lax.fori_loop(0, _UNIT, zfill, 0)
        _zero_acc_slice(rows0, acc, base)
        plsc.subcore_barrier()

        def gather(u, buf, sem):
            return pltpu.async_copy(h.at[srcall.at[u]], buf, sem)

        def gwait(u, buf, sem):
            pltpu.make_async_copy(h.at[srcall.at[u]], buf, sem).wait()

        def scatter(u, buf):
            pltpu.sync_copy(buf, acc.at[dstall.at[u]], add=True)

        for p in range(2):
            pltpu.sync_copy(src2d.at[pl.ds(u0 + p * _UPP, _UPP)], srcall)
            pltpu.sync_copy(dst2d.at[pl.ds(u0 + p * _UPP, _UPP)], dstall)
            gather(0, rows0, g0)

            def step(i, carry):
                u = 2 * i
                gather(u + 1, rows1, g1)
                gwait(u, rows0, g0)
                scatter(u, rows0)

                @pl.when(u + 2 < _UPP)
                def _():
                    gather(u + 2, rows0, g0)

                gwait(u + 1, rows1, g1)
                scatter(u + 1, rows1)
                return carry

            lax.fori_loop(0, _UPP // 2, step, 0)

        plsc.subcore_barrier()
        pltpu.sync_copy(acc.at[pl.ds(base, _RPS)],
                        out.at[cid, pl.ds(base, _RPS)])

    return agg_kernel


_deg_call = _make_deg()
_agg_h = _make_agg(_H)

_BLK = 1000
_GRID = _N // _BLK


def _dinv_of(d_ref):
    s = d_ref[0, :, 0:1] + d_ref[1, :, 0:1] + 1.0
    return lax.rsqrt(s)


def _mm1(x, W1, deg):
    def body(x_ref, w_ref, d_ref, o_ref):
        dinv = _dinv_of(d_ref)
        h = jnp.dot(x_ref[...], w_ref[...], preferred_element_type=jnp.float32)
        o_ref[...] = h * dinv

    return pl.pallas_call(
        body,
        grid=(_GRID,),
        in_specs=[
            pl.BlockSpec((_BLK, _D), lambda i: (i, 0)),
            pl.BlockSpec((_D, _H), lambda i: (0, 0)),
            pl.BlockSpec((_NC, _BLK, _DEGW), lambda i: (0, i, 0)),
        ],
        out_specs=pl.BlockSpec((_BLK, _H), lambda i: (i, 0)),
        out_shape=jax.ShapeDtypeStruct((_N, _H), jnp.float32),
    )(x, W1, deg)


def _mm2(acc1, h1s, deg, x, b1, W2):
    def body(a_ref, h_ref, d_ref, x_ref, b_ref, w_ref, o_ref):
        dinv = _dinv_of(d_ref)
        agg = a_ref[0] + a_ref[1] + h_ref[...]
        out1 = agg * dinv + b_ref[...]
        z = jnp.maximum(out1, 0.0) + x_ref[...]
        h2 = jnp.dot(z, w_ref[...], preferred_element_type=jnp.float32)
        o_ref[...] = jnp.concatenate([h2 * dinv, jnp.zeros_like(h2)], axis=1)

    return pl.pallas_call(
        body,
        grid=(_GRID,),
        in_specs=[
            pl.BlockSpec((_NC, _BLK, _H), lambda i: (0, i, 0)),
            pl.BlockSpec((_BLK, _H), lambda i: (i, 0)),
            pl.BlockSpec((_NC, _BLK, _DEGW), lambda i: (0, i, 0)),
            pl.BlockSpec((_BLK, _D), lambda i: (i, 0)),
            pl.BlockSpec((_H,), lambda i: (0,)),
            pl.BlockSpec((_H, _C), lambda i: (0, 0)),
        ],
        out_specs=pl.BlockSpec((_BLK, 2 * _C), lambda i: (i, 0)),
        out_shape=jax.ShapeDtypeStruct((_N, 2 * _C), jnp.float32),
    )(acc1, h1s, deg, x, b1, W2)


def _mm3(acc2, h2s, deg, b2):
    def body(a_ref, h_ref, d_ref, b_ref, o_ref):
        dinv = _dinv_of(d_ref)
        agg = a_ref[0, :, : _C] + a_ref[1, :, : _C] + h_ref[:, : _C]
        o_ref[...] = agg * dinv + b_ref[...]

    return pl.pallas_call(
        body,
        grid=(_GRID,),
        in_specs=[
            pl.BlockSpec((_NC, _BLK, 2 * _C), lambda i: (0, i, 0)),
            pl.BlockSpec((_BLK, 2 * _C), lambda i: (i, 0)),
            pl.BlockSpec((_NC, _BLK, _DEGW), lambda i: (0, i, 0)),
            pl.BlockSpec((_C,), lambda i: (0,)),
        ],
        out_specs=pl.BlockSpec((_BLK, _C), lambda i: (i, 0)),
        out_shape=jax.ShapeDtypeStruct((_N, _C), jnp.float32),
    )(acc2, h2s, deg, b2)


def kernel(x, edge_index, W1, b1, W2, b2):
    ei = edge_index.astype(jnp.int32)
    srcs = ei[0]
    dsts = ei[1]
    npad = _NUNITS_PAD * _UNIT - _E
    pad_src = (jnp.arange(npad, dtype=jnp.int32) * 131) % _N
    src2d = jnp.concatenate([srcs, pad_src]).reshape(_NUNITS_PAD, _UNIT)
    dst2d = jnp.concatenate(
        [dsts, _N + (jnp.arange(npad, dtype=jnp.int32) % (_NP - _N))]
    ).reshape(_NUNITS_PAD, _UNIT)
    deg = _deg_call(dst2d)
    h1s = _mm1(x, W1, deg)
    acc1 = _agg_h(src2d, dst2d, h1s)
    h2s = _mm2(acc1, h1s, deg, x, b1, W2)
    acc2 = _agg_h(src2d, dst2d, h2s)
    y = _mm3(acc2, h2s, deg, b2)
    return y


# speedup vs baseline: 3.1614x; 1.0022x over previous
import functools

import jax
import jax.numpy as jnp
from jax import lax
from jax.experimental import pallas as pl
from jax.experimental.pallas import tpu as pltpu
from jax.experimental.pallas import tpu_sc as plsc

_N, _E, _D, _H, _C = 10000, 320000, 128, 128, 64
_NC, _NS = 2, 16
_NW = _NC * _NS
_UNIT = 128
_NUNITS = _E // _UNIT
_BASE = _NUNITS // _NW
_EXTRA = _NUNITS - _BASE * _NW
_NUNITS_PAD = 2560
_NP = 10240
_RPS = _NP // _NS
_DEGW = 128
_DEGS = 8

_mesh = lambda: plsc.VectorSubcoreMesh(core_axis_name="c", subcore_axis_name="s")


def _zero16():
    return jnp.zeros((16,), jnp.float32)


def _zero_acc_slice(zrows, acc, base):
    for t in range(_RPS // _UNIT):
        pltpu.sync_copy(zrows, acc.at[pl.ds(base + t * _UNIT, _UNIT)])


def _make_deg():
    @functools.partial(
        pl.kernel,
        out_type=jax.ShapeDtypeStruct((_NC, _NP, _DEGW), jnp.float32),
        mesh=_mesh(),
        scratch_types=[
            pltpu.VMEM((40, _UNIT), jnp.int32),
            pltpu.VMEM((_UNIT, _DEGW), jnp.float32),
            pltpu.VMEM((_UNIT, _DEGW), jnp.float32),
            pltpu.SemaphoreType.DMA,
            pltpu.SemaphoreType.DMA,
            pltpu.VMEM_SHARED((_NP, _DEGW), jnp.float32),
        ],
    )
    def deg_kernel(dst2d, out, dstall, onesv, zv, s0, s1, acc):
        cid = lax.axis_index("c")
        sid = lax.axis_index("s")
        wid = sid * _NC + cid
        base = sid * _RPS
        u0 = wid * _UPW

        ones16 = jnp.ones((16,), jnp.float32)
        z16 = _zero16()

        def fill(r, carry):
            for cidx in range(_DEGW // 16):
                onesv[r, pl.ds(cidx * 16, 16)] = ones16
                zv[r, pl.ds(cidx * 16, 16)] = z16
            return carry

        lax.fori_loop(0, _UNIT, fill, 0)
        _zero_acc_slice(zv, acc, base)
        plsc.subcore_barrier()

        def scat(u, sem):
            pltpu.async_copy(onesv, acc.at[dstall.at[u]], sem, add=True)

        def swait(sem):
            pltpu.make_async_copy(onesv, acc.at[dstall.at[0]], sem).wait()

        for p in range(2):
            pltpu.sync_copy(dst2d.at[pl.ds(u0 + p * _UPP, _UPP)], dstall)
            scat(0, s0)
            scat(1, s1)

            def step(i, carry):
                u = 2 * i
                swait(s0)

                @pl.when(u + 2 < _UPP)
                def _():
                    scat(u + 2, s0)

                swait(s1)

                @pl.when(u + 3 < _UPP)
                def _():
                    scat(u + 3, s1)

                return carry

            lax.fori_loop(0, _UPP // 2, step, 0)

        plsc.subcore_barrier()
        pltpu.sync_copy(acc.at[pl.ds(base, _RPS)],
                        out.at[cid, pl.ds(base, _RPS)])

    return deg_kernel


_UPW = _NUNITS_PAD // _NW
_UPP = _UPW // 2


def _make_agg(F):
    @functools.partial(
        pl.kernel,
        out_type=jax.ShapeDtypeStruct((_NC, _NP, F), jnp.float32),
        mesh=_mesh(),
        scratch_types=[
            pltpu.VMEM((_UPP, _UNIT), jnp.int32),
            pltpu.VMEM((_UPP, _UNIT), jnp.int32),
            pltpu.VMEM((_UNIT, F), jnp.float32),
            pltpu.VMEM((_UNIT, F), jnp.float32),
            pltpu.SemaphoreType.DMA,
            pltpu.SemaphoreType.DMA,
            pltpu.VMEM_SHARED((_NP, F), jnp.float32),
        ],
    )
    def agg_kernel(src2d, dst2d, h, out, srcall, dstall, rows0, rows1,
                   g0, g1, acc):
        cid = lax.axis_index("c")
        sid = lax.axis_index("s")
        wid = sid * _NC + cid
        base = sid * _RPS
        u0 = wid * _UPW

        z16 = _zero16()

        def zfill(r, carry):
            for cidx in range(F // 16):
                rows0[r, pl.ds(cidx * 16, 16)] = z16
            return carry

        lax.fori_loop(0, _UNIT, zfill, 0)
        _zero_acc_slice(rows0, acc, base)
        plsc.subcore_barrier()

        def gather(u, buf, sem):
            return pltpu.async_copy(h.at[srcall.at[u]], buf, sem)

        def gwait(u, buf, sem):
            pltpu.make_async_copy(h.at[srcall.at[u]], buf, sem).wait()

        def scatter(u, buf):
            pltpu.sync_copy(buf, acc.at[dstall.at[u]], add=True)

        for p in range(2):
            pltpu.sync_copy(src2d.at[pl.ds(u0 + p * _UPP, _UPP)], srcall)
            pltpu.sync_copy(dst2d.at[pl.ds(u0 + p * _UPP, _UPP)], dstall)
            gather(0, rows0, g0)

            def step(i, carry):
                u = 2 * i
                gather(u + 1, rows1, g1)
                gwait(u, rows0, g0)
                scatter(u, rows0)

                @pl.when(u + 2 < _UPP)
                def _():
                    gather(u + 2, rows0, g0)

                gwait(u + 1, rows1, g1)
                scatter(u + 1, rows1)
                return carry

            lax.fori_loop(0, _UPP // 2, step, 0)

        plsc.subcore_barrier()
        pltpu.sync_copy(acc.at[pl.ds(base, _RPS)],
                        out.at[cid, pl.ds(base, _RPS)])

    return agg_kernel


_deg_call = _make_deg()
_agg_h = _make_agg(_H)

_BLK = 1000
_GRID = _N // _BLK


def _dinv_of(d_ref):
    s = d_ref[0, :, 0:1] + d_ref[1, :, 0:1] + 1.0
    return lax.rsqrt(s)


def _mm1raw(x, W1):
    def body(x_ref, w_ref, o_ref):
        o_ref[...] = jnp.dot(x_ref[...], w_ref[...],
                             preferred_element_type=jnp.float32)

    return pl.pallas_call(
        body,
        grid=(_GRID,),
        in_specs=[
            pl.BlockSpec((_BLK, _D), lambda i: (i, 0)),
            pl.BlockSpec((_D, _H), lambda i: (0, 0)),
        ],
        out_specs=pl.BlockSpec((_BLK, _H), lambda i: (i, 0)),
        out_shape=jax.ShapeDtypeStruct((_N, _H), jnp.float32),
    )(x, W1)


def _scale1(h1r, deg):
    def body(h_ref, d_ref, o_ref):
        o_ref[...] = h_ref[...] * _dinv_of(d_ref)

    return pl.pallas_call(
        body,
        grid=(_GRID,),
        in_specs=[
            pl.BlockSpec((_BLK, _H), lambda i: (i, 0)),
            pl.BlockSpec((_NC, _BLK, _DEGS), lambda i: (0, i, 0)),
        ],
        out_specs=pl.BlockSpec((_BLK, _H), lambda i: (i, 0)),
        out_shape=jax.ShapeDtypeStruct((_N, _H), jnp.float32),
    )(h1r, deg)


def _mm2(acc1, h1s, deg, x, b1, W2):
    def body(a_ref, h_ref, d_ref, x_ref, b_ref, w_ref, o_ref):
        dinv = _dinv_of(d_ref)
        agg = a_ref[0] + a_ref[1] + h_ref[...]
        out1 = agg * dinv + b_ref[...]
        z = jnp.maximum(out1, 0.0) + x_ref[...]
        h2 = jnp.dot(z, w_ref[...], preferred_element_type=jnp.float32)
        o_ref[...] = jnp.concatenate([h2 * dinv, jnp.zeros_like(h2)], axis=1)

    return pl.pallas_call(
        body,
        grid=(_GRID,),
        in_specs=[
            pl.BlockSpec((_NC, _BLK, _H), lambda i: (0, i, 0)),
            pl.BlockSpec((_BLK, _H), lambda i: (i, 0)),
            pl.BlockSpec((_NC, _BLK, _DEGS), lambda i: (0, i, 0)),
            pl.BlockSpec((_BLK, _D), lambda i: (i, 0)),
            pl.BlockSpec((_H,), lambda i: (0,)),
            pl.BlockSpec((_H, _C), lambda i: (0, 0)),
        ],
        out_specs=pl.BlockSpec((_BLK, 2 * _C), lambda i: (i, 0)),
        out_shape=jax.ShapeDtypeStruct((_N, 2 * _C), jnp.float32),
    )(acc1, h1s, deg, x, b1, W2)


def _mm3(acc2, h2s, deg, b2):
    def body(a_ref, h_ref, d_ref, b_ref, o_ref):
        dinv = _dinv_of(d_ref)
        agg = a_ref[0, :, : _C] + a_ref[1, :, : _C] + h_ref[:, : _C]
        o_ref[...] = agg * dinv + b_ref[...]

    return pl.pallas_call(
        body,
        grid=(_GRID,),
        in_specs=[
            pl.BlockSpec((_NC, _BLK, 2 * _C), lambda i: (0, i, 0)),
            pl.BlockSpec((_BLK, 2 * _C), lambda i: (i, 0)),
            pl.BlockSpec((_NC, _BLK, _DEGS), lambda i: (0, i, 0)),
            pl.BlockSpec((_C,), lambda i: (0,)),
        ],
        out_specs=pl.BlockSpec((_BLK, _C), lambda i: (i, 0)),
        out_shape=jax.ShapeDtypeStruct((_N, _C), jnp.float32),
    )(acc2, h2s, deg, b2)


def kernel(x, edge_index, W1, b1, W2, b2):
    ei = edge_index.astype(jnp.int32)
    srcs = ei[0]
    dsts = ei[1]
    npad = _NUNITS_PAD * _UNIT - _E
    pad_src = (jnp.arange(npad, dtype=jnp.int32) * 131) % _N
    src2d = jnp.concatenate([srcs, pad_src]).reshape(_NUNITS_PAD, _UNIT)
    dst2d = jnp.concatenate(
        [dsts, _N + (jnp.arange(npad, dtype=jnp.int32) % (_NP - _N))]
    ).reshape(_NUNITS_PAD, _UNIT)
    h1r = _mm1raw(x, W1)
    degw = _deg_call(dst2d)
    deg = jax.lax.slice(degw, (0, 0, 0), (_NC, _NP, _DEGS))
    h1s = _scale1(h1r, deg)
    acc1 = _agg_h(src2d, dst2d, h1s)
    h2s = _mm2(acc1, h1s, deg, x, b1, W2)
    acc2 = _agg_h(src2d, dst2d, h2s)
    y = _mm3(acc2, h2s, deg, b2)
    return y
